# Initial kernel scaffold; baseline (speedup 1.0000x reference)
import functools

import jax
import jax.numpy as jnp
from jax import lax
from jax.experimental import pallas as pl
from jax.experimental.pallas import tpu as pltpu
from jax.experimental.pallas import tpu_sc as plsc

N_NODES = 100000
N_EDGES = 6400000
NUM_TYPES = 64

_INFO = plsc.get_sparse_core_info()
_NC = _INFO.num_cores
_NS = _INFO.num_subcores
_NW = _NC * _NS
_L = _INFO.num_lanes

_E_PER_W = N_EDGES // _NW
_CHUNK = 2000
_N_CHUNKS = _E_PER_W // _CHUNK


def _sc_body(ei_hbm, types_hbm, eng_hbm, scales_hbm, shifts_hbm, out_hbm,
             types_v, scales_v, shifts_v, cidx_v, nidx_v, eng_v):
    wid = lax.axis_index("s") * _NC + lax.axis_index("c")
    wstart = wid * _E_PER_W

    pltpu.sync_copy(types_hbm, types_v)
    pltpu.sync_copy(scales_hbm, scales_v)
    pltpu.sync_copy(shifts_hbm, shifts_v)

    def chunk_body(i, carry):
        base = wstart + i * _CHUNK
        pltpu.sync_copy(ei_hbm.at[pl.ds(base, _CHUNK)], cidx_v)
        pltpu.sync_copy(ei_hbm.at[pl.ds(N_EDGES + base, _CHUNK)], nidx_v)
        pltpu.sync_copy(eng_hbm.at[pl.ds(base, _CHUNK)], eng_v)

        def lane_body(j, c2):
            off = j * _L
            c = cidx_v[pl.ds(off, _L)]
            n = nidx_v[pl.ds(off, _L)]
            cs = plsc.load_gather(types_v, [c])
            ns = plsc.load_gather(types_v, [n])
            flat = cs * NUM_TYPES + ns
            sc = plsc.load_gather(scales_v, [flat])
            sh = plsc.load_gather(shifts_v, [flat])
            x = eng_v[pl.ds(off, _L)]
            eng_v[pl.ds(off, _L)] = sc * x + sh
            return c2

        lax.fori_loop(0, _CHUNK // _L, lane_body, 0)
        pltpu.sync_copy(eng_v, out_hbm.at[pl.ds(base, _CHUNK)])
        return carry

    lax.fori_loop(0, _N_CHUNKS, chunk_body, 0)


_mesh = plsc.VectorSubcoreMesh(core_axis_name="c", subcore_axis_name="s")

_sc_kernel = functools.partial(
    pl.kernel,
    mesh=_mesh,
    out_type=jax.ShapeDtypeStruct((N_EDGES,), jnp.float32),
    scratch_types=[
        pltpu.VMEM((N_NODES,), jnp.int32),
        pltpu.VMEM((NUM_TYPES * NUM_TYPES,), jnp.float32),
        pltpu.VMEM((NUM_TYPES * NUM_TYPES,), jnp.float32),
        pltpu.VMEM((_CHUNK,), jnp.int32),
        pltpu.VMEM((_CHUNK,), jnp.int32),
        pltpu.VMEM((_CHUNK,), jnp.float32),
    ],
)(_sc_body)


def kernel(edge_index, atom_types, edge_energy, scales, shifts):
    ei_flat = edge_index.reshape(-1)
    eng_flat = edge_energy.reshape(-1)
    scales_flat = scales.reshape(-1)
    shifts_flat = shifts.reshape(-1)
    out = _sc_kernel(ei_flat, atom_types, eng_flat, scales_flat, shifts_flat)
    return out.reshape(-1, 1)


# speedup vs baseline: 557.8035x; 557.8035x over previous
import functools

import jax
import jax.numpy as jnp
from jax import lax
from jax.experimental import pallas as pl
from jax.experimental.pallas import tpu as pltpu
from jax.experimental.pallas import tpu_sc as plsc

N_NODES = 100000
N_EDGES = 6400000
NUM_TYPES = 64

_INFO = plsc.get_sparse_core_info()
_NC = _INFO.num_cores
_NS = _INFO.num_subcores
_NW = _NC * _NS
_L = _INFO.num_lanes

_E_PER_W = N_EDGES // _NW
_CHUNK = 2000
_N_CHUNKS = _E_PER_W // _CHUNK


def _sc_body(ei_hbm, types_hbm, eng_hbm, scales_hbm, shifts_hbm, out_hbm,
             types_v, scales_v, shifts_v, cidx_v, nidx_v, eng_v):
    wid = lax.axis_index("s") * _NC + lax.axis_index("c")
    wstart = wid * _E_PER_W

    pltpu.sync_copy(types_hbm, types_v)
    pltpu.sync_copy(scales_hbm, scales_v)
    pltpu.sync_copy(shifts_hbm, shifts_v)

    def chunk_body(i, carry):
        base = wstart + i * _CHUNK
        pltpu.sync_copy(ei_hbm.at[pl.ds(base, _CHUNK)], cidx_v)
        pltpu.sync_copy(ei_hbm.at[pl.ds(N_EDGES + base, _CHUNK)], nidx_v)
        pltpu.sync_copy(eng_hbm.at[pl.ds(base, _CHUNK)], eng_v)

        def lane_body(j, c2):
            off = j * _L
            c = cidx_v[pl.ds(off, _L)]
            n = nidx_v[pl.ds(off, _L)]
            cs = plsc.load_gather(types_v, [c])
            ns = plsc.load_gather(types_v, [n])
            flat = cs * NUM_TYPES + ns
            sc = plsc.load_gather(scales_v, [flat])
            sh = plsc.load_gather(shifts_v, [flat])
            x = eng_v[pl.ds(off, _L)]
            eng_v[pl.ds(off, _L)] = sc * x + sh
            return c2

        lax.fori_loop(0, _CHUNK // _L, lane_body, 0)
        pltpu.sync_copy(eng_v, out_hbm.at[pl.ds(base, _CHUNK)])
        return carry

    lax.fori_loop(0, _N_CHUNKS, chunk_body, 0)


_mesh = plsc.VectorSubcoreMesh(core_axis_name="c", subcore_axis_name="s")

_sc_kernel = functools.partial(
    pl.kernel,
    mesh=_mesh,
    out_type=jax.ShapeDtypeStruct((N_EDGES,), jnp.float32),
    compiler_params=pltpu.CompilerParams(needs_layout_passes=False),
    scratch_types=[
        pltpu.VMEM((N_NODES,), jnp.int32),
        pltpu.VMEM((NUM_TYPES * NUM_TYPES,), jnp.float32),
        pltpu.VMEM((NUM_TYPES * NUM_TYPES,), jnp.float32),
        pltpu.VMEM((_CHUNK,), jnp.int32),
        pltpu.VMEM((_CHUNK,), jnp.int32),
        pltpu.VMEM((_CHUNK,), jnp.float32),
    ],
)(_sc_body)


def kernel(edge_index, atom_types, edge_energy, scales, shifts):
    ei_flat = edge_index.reshape(-1)
    eng_flat = edge_energy.reshape(-1)
    scales_flat = scales.reshape(-1)
    shifts_flat = shifts.reshape(-1)
    out = _sc_kernel(ei_flat, atom_types, eng_flat, scales_flat, shifts_flat)
    return out.reshape(-1, 1)


# speedup vs baseline: 1683.3912x; 3.0179x over previous
import functools

import jax
import jax.numpy as jnp
from jax import lax
from jax.experimental import pallas as pl
from jax.experimental.pallas import tpu as pltpu
from jax.experimental.pallas import tpu_sc as plsc

N_NODES = 100000
N_EDGES = 6400000
NUM_TYPES = 64

_INFO = plsc.get_sparse_core_info()
_NC = _INFO.num_cores
_NS = _INFO.num_subcores
_NW = _NC * _NS
_L = _INFO.num_lanes

_E_PER_W = N_EDGES // _NW
_CHUNK = 2000
_N_CHUNKS = _E_PER_W // _CHUNK


_NBUF = 2
_UNROLL = 5


def _sc_body(ei_hbm, types_hbm, eng_hbm, scales_hbm, shifts_hbm, out_hbm,
             types_v, scales_v, shifts_v, cidx0, cidx1, nidx0, nidx1,
             eng0, eng1, outv0, outv1, in_sem0, in_sem1, out_sem0, out_sem1):
    wid = lax.axis_index("s") * _NC + lax.axis_index("c")
    wstart = wid * _E_PER_W
    cidx_b = (cidx0, cidx1)
    nidx_b = (nidx0, nidx1)
    eng_b = (eng0, eng1)
    outv_b = (outv0, outv1)
    in_sems = (in_sem0, in_sem1)
    out_sems = (out_sem0, out_sem1)

    pltpu.sync_copy(types_hbm, types_v)
    pltpu.sync_copy(scales_hbm, scales_v)
    pltpu.sync_copy(shifts_hbm, shifts_v)

    def start_in(base, b):
        pltpu.async_copy(ei_hbm.at[pl.ds(base, _CHUNK)], cidx_b[b], in_sems[b])
        pltpu.async_copy(ei_hbm.at[pl.ds(N_EDGES + base, _CHUNK)], nidx_b[b],
                         in_sems[b])
        pltpu.async_copy(eng_hbm.at[pl.ds(base, _CHUNK)], eng_b[b], in_sems[b])

    def wait_in(b):
        pltpu.make_async_copy(ei_hbm.at[pl.ds(0, _CHUNK)], cidx_b[b],
                              in_sems[b]).wait()
        pltpu.make_async_copy(ei_hbm.at[pl.ds(0, _CHUNK)], nidx_b[b],
                              in_sems[b]).wait()
        pltpu.make_async_copy(eng_hbm.at[pl.ds(0, _CHUNK)], eng_b[b],
                              in_sems[b]).wait()

    def wait_out(b):
        pltpu.make_async_copy(outv_b[b], out_hbm.at[pl.ds(0, _CHUNK)],
                              out_sems[b]).wait()

    start_in(wstart, 0)
    start_in(wstart + _CHUNK, 1)

    def outer(io, carry):
        for b in range(_NBUF):
            i = io * _NBUF + b
            base = wstart + i * _CHUNK
            wait_in(b)

            @pl.when(i >= _NBUF)
            def _():
                wait_out(b)

            cidx = cidx_b[b]
            nidx = nidx_b[b]
            eng = eng_b[b]
            outv = outv_b[b]

            @plsc.parallel_loop(0, _CHUNK // _L, 1, unroll=_UNROLL)
            def lane_body(j):
                off = j * _L
                c = cidx[pl.ds(off, _L)]
                n = nidx[pl.ds(off, _L)]
                cs = plsc.load_gather(types_v, [c])
                ns = plsc.load_gather(types_v, [n])
                flat = cs * NUM_TYPES + ns
                sc = plsc.load_gather(scales_v, [flat])
                sh = plsc.load_gather(shifts_v, [flat])
                x = eng[pl.ds(off, _L)]
                outv[pl.ds(off, _L)] = sc * x + sh

            pltpu.async_copy(outv, out_hbm.at[pl.ds(base, _CHUNK)], out_sems[b])

            @pl.when(i + _NBUF < _N_CHUNKS)
            def _():
                start_in(base + _NBUF * _CHUNK, b)

        return carry

    lax.fori_loop(0, _N_CHUNKS // _NBUF, outer, 0)
    wait_out(0)
    wait_out(1)


_mesh = plsc.VectorSubcoreMesh(core_axis_name="c", subcore_axis_name="s")

_sc_kernel = functools.partial(
    pl.kernel,
    mesh=_mesh,
    out_type=jax.ShapeDtypeStruct((N_EDGES,), jnp.float32),
    compiler_params=pltpu.CompilerParams(needs_layout_passes=False),
    scratch_types=[
        pltpu.VMEM((N_NODES,), jnp.int32),
        pltpu.VMEM((NUM_TYPES * NUM_TYPES,), jnp.float32),
        pltpu.VMEM((NUM_TYPES * NUM_TYPES,), jnp.float32),
        pltpu.VMEM((_CHUNK,), jnp.int32),
        pltpu.VMEM((_CHUNK,), jnp.int32),
        pltpu.VMEM((_CHUNK,), jnp.int32),
        pltpu.VMEM((_CHUNK,), jnp.int32),
        pltpu.VMEM((_CHUNK,), jnp.float32),
        pltpu.VMEM((_CHUNK,), jnp.float32),
        pltpu.VMEM((_CHUNK,), jnp.float32),
        pltpu.VMEM((_CHUNK,), jnp.float32),
        pltpu.SemaphoreType.DMA,
        pltpu.SemaphoreType.DMA,
        pltpu.SemaphoreType.DMA,
        pltpu.SemaphoreType.DMA,
    ],
)(_sc_body)


def kernel(edge_index, atom_types, edge_energy, scales, shifts):
    ei_flat = edge_index.reshape(-1)
    eng_flat = edge_energy.reshape(-1)
    scales_flat = scales.reshape(-1)
    shifts_flat = shifts.reshape(-1)
    out = _sc_kernel(ei_flat, atom_types, eng_flat, scales_flat, shifts_flat)
    return out.reshape(-1, 1)
